# baseline (speedup 1.0000x reference)
import jax
import jax.numpy as jnp
from jax.experimental import pallas as pl
from jax.experimental.pallas import tpu as pltpu

_N1 = 2048
_N2 = 2048
_D = 16
_H = 1024


def _acosh(v):
    return jnp.log(v + jnp.sqrt(v * v - 1.0))


def _aug_x(x):
    xn = jnp.sum(x * x, axis=1, keepdims=True)
    c = 2.0 / (1.0 - xn)
    return jnp.concatenate([x * (-2.0 * c), xn * c, c], axis=1)


def _aug_y(y):
    yn = jnp.sum(y * y, axis=1, keepdims=True)
    b = 1.0 / (1.0 - yn)
    return jnp.concatenate([y * b, b, yn * b], axis=1)


def _mm(a, b):
    return jax.lax.dot_general(
        a, b, (((1,), (1,)), ((), ())), preferred_element_type=jnp.float32)


def _hausdorff_kernel(x_hbm, y_hbm, out_ref, x_v, y_v, sx, sy0, sy1):
    cx = pltpu.make_async_copy(x_hbm, x_v, sx)
    cy0 = pltpu.make_async_copy(y_hbm.at[pl.ds(0, _H), :], y_v.at[pl.ds(0, _H), :], sy0)
    cy1 = pltpu.make_async_copy(y_hbm.at[pl.ds(_H, _H), :], y_v.at[pl.ds(_H, _H), :], sy1)
    cx.start()
    cy0.start()
    cy1.start()

    cx.wait()
    ax = _aug_x(x_v[...])

    cy0.wait()
    ay0 = _aug_y(y_v[pl.ds(0, _H), :])
    m0 = _mm(ax, ay0)
    r = jnp.min(m0, axis=1, keepdims=True)
    c0 = jnp.min(m0, axis=0, keepdims=True)

    cy1.wait()
    ay1 = _aug_y(y_v[pl.ds(_H, _H), :])
    m1 = _mm(ax, ay1)
    r = jnp.minimum(r, jnp.min(m1, axis=1, keepdims=True))
    c1 = jnp.min(m1, axis=0, keepdims=True)

    rsum = jnp.sum(_acosh(1.0 + r))
    csum = jnp.sum(_acosh(1.0 + c0)) + jnp.sum(_acosh(1.0 + c1))
    out_ref[...] = jnp.reshape(rsum / _N1 + csum / _N2, (1, 1))


def kernel(set1, set2):
    out = pl.pallas_call(
        _hausdorff_kernel,
        out_shape=jax.ShapeDtypeStruct((1, 1), jnp.float32),
        in_specs=[
            pl.BlockSpec(memory_space=pl.ANY),
            pl.BlockSpec(memory_space=pl.ANY),
        ],
        out_specs=pl.BlockSpec(memory_space=pltpu.VMEM),
        scratch_shapes=[
            pltpu.VMEM((_N1, _D), jnp.float32),
            pltpu.VMEM((_N2, _D), jnp.float32),
            pltpu.SemaphoreType.DMA,
            pltpu.SemaphoreType.DMA,
            pltpu.SemaphoreType.DMA,
        ],
    )(set1, set2)
    return out[0, 0]


# speedup vs baseline: 1.0618x; 1.0618x over previous
import jax
import jax.numpy as jnp
from jax.experimental import pallas as pl
from jax.experimental.pallas import tpu as pltpu

_N1 = 2048
_N2 = 2048
_D = 16
_H = 1024


def _acosh(v):
    return jnp.log(v + jnp.sqrt(v * v - 1.0))


def _aug_x(x):
    xn = jnp.sum(x * x, axis=1, keepdims=True)
    c = 2.0 / (1.0 - xn)
    return jnp.concatenate([x * (-2.0 * c), xn * c, c], axis=1)


def _aug_y(y):
    yn = jnp.sum(y * y, axis=1, keepdims=True)
    b = 1.0 / (1.0 - yn)
    return jnp.concatenate([y * b, b, yn * b], axis=1)


def _mm(a, b):
    return jax.lax.dot_general(
        a, b, (((1,), (1,)), ((), ())), preferred_element_type=jnp.float32)


def _hausdorff_kernel(x_hbm, y_hbm, out_ref, x_v, y_v, sx, sy):
    cx = pltpu.make_async_copy(x_hbm, x_v, sx)
    cy = pltpu.make_async_copy(y_hbm, y_v, sy)
    cx.start()
    cy.start()

    cx.wait()
    ax = _aug_x(x_v[...])

    cy.wait()
    ay = _aug_y(y_v[...])
    m = _mm(ax, ay)
    r = jnp.min(m, axis=1, keepdims=True)
    c0 = jnp.min(m, axis=0, keepdims=True)

    rsum = jnp.sum(_acosh(1.0 + r))
    csum = jnp.sum(_acosh(1.0 + c0))
    out_ref[...] = jnp.reshape(rsum / _N1 + csum / _N2, (1, 1))


def kernel(set1, set2):
    out = pl.pallas_call(
        _hausdorff_kernel,
        out_shape=jax.ShapeDtypeStruct((1, 1), jnp.float32),
        in_specs=[
            pl.BlockSpec(memory_space=pl.ANY),
            pl.BlockSpec(memory_space=pl.ANY),
        ],
        out_specs=pl.BlockSpec(memory_space=pltpu.VMEM),
        scratch_shapes=[
            pltpu.VMEM((_N1, _D), jnp.float32),
            pltpu.VMEM((_N2, _D), jnp.float32),
            pltpu.SemaphoreType.DMA,
            pltpu.SemaphoreType.DMA,
        ],
    )(set1, set2)
    return out[0, 0]


# speedup vs baseline: 1.0799x; 1.0171x over previous
import jax
import jax.numpy as jnp
from jax.experimental import pallas as pl
from jax.experimental.pallas import tpu as pltpu

_N1 = 2048
_N2 = 2048
_D = 16


def _acosh(v):
    return jnp.log(v + jnp.sqrt(v * v - 1.0))


def _hausdorff_kernel(x_ref, y_ref, out_ref):
    x = x_ref[...]
    y = y_ref[...]
    xn = jnp.sum(x * x, axis=1, keepdims=True)
    yn = jnp.sum(y * y, axis=1, keepdims=True)
    c = 2.0 / (1.0 - xn)
    b = 1.0 / (1.0 - yn)
    ax = jnp.concatenate([x * (-2.0 * c), xn * c, c], axis=1)
    ay = jnp.concatenate([y * b, b, yn * b], axis=1)
    m = jax.lax.dot_general(
        ax, ay, (((1,), (1,)), ((), ())),
        preferred_element_type=jnp.float32)
    rmin = 1.0 + jnp.min(m, axis=1, keepdims=True)
    cmin = 1.0 + jnp.min(m, axis=0, keepdims=True)
    total = jnp.sum(_acosh(rmin)) / _N1 + jnp.sum(_acosh(cmin)) / _N2
    out_ref[...] = jnp.reshape(total, (1, 1))


def kernel(set1, set2):
    out = pl.pallas_call(
        _hausdorff_kernel,
        out_shape=jax.ShapeDtypeStruct((1, 1), jnp.float32),
        in_specs=[
            pl.BlockSpec(memory_space=pltpu.VMEM),
            pl.BlockSpec(memory_space=pltpu.VMEM),
        ],
        out_specs=pl.BlockSpec(memory_space=pltpu.VMEM),
    )(set1, set2)
    return out[0, 0]
